# baseline (speedup 1.0000x reference)
import functools
import math

import jax
import jax.numpy as jnp
from jax import lax
from jax.experimental import pallas as pl
from jax.experimental.pallas import tpu as pltpu
from jax.experimental.pallas import tpu_sc as plsc

N0 = 10000
E0 = 160000
NPAD = 10240
TRASH = N0
EP = 163840
BM = 256
KB = 128
NSUB = 16
RPT = NPAD // NSUB


def _leaky(x):
    return jnp.where(x >= 0, x, 0.01 * x)


def _linear_body(*refs, nparts, act):
    out_ref = refs[-1]
    bias_ref = refs[-2]
    acc = jnp.zeros(out_ref.shape, jnp.float32)
    for i in range(nparts):
        a = refs[2 * i][...]
        w = refs[2 * i + 1][...]
        acc += jnp.dot(a, w, preferred_element_type=jnp.float32)
    acc = acc + bias_ref[...]
    if act:
        acc = _leaky(acc)
    out_ref[...] = acc


def _linear(parts, bias, act):
    rows = parts[0][0].shape[0]
    dout = parts[0][1].shape[1]
    bm = BM if rows % BM == 0 else rows
    grid = (rows // bm,)
    in_specs = []
    args = []
    for a, w in parts:
        di = a.shape[1]
        in_specs.append(pl.BlockSpec((bm, di), lambda i: (i, 0)))
        in_specs.append(pl.BlockSpec((di, dout), lambda i: (0, 0)))
        args += [a, w]
    b2 = jnp.zeros((1, dout), jnp.float32) if bias is None else bias.reshape(1, dout)
    in_specs.append(pl.BlockSpec((1, dout), lambda i: (0, 0)))
    args.append(b2)
    return pl.pallas_call(
        functools.partial(_linear_body, nparts=len(parts), act=act),
        grid=grid,
        in_specs=in_specs,
        out_specs=pl.BlockSpec((bm, dout), lambda i: (i, 0)),
        out_shape=jax.ShapeDtypeStruct((rows, dout), jnp.float32),
    )(*args)


def _head_body(r1, r2, r3, sc, lw, lb, w1, b1, g1, e1, w2, b2, g2, e2,
               w3, b3, g3, e3, cw, cb, out):
    bnc = 1.0 / math.sqrt(1.0 + 1e-5)
    llm = jnp.dot(sc[...], lw[...], preferred_element_type=jnp.float32) + lb[...]
    h = jnp.concatenate([r1[...], r2[...], r3[...], llm], axis=1)
    h = jnp.dot(h, w1[...], preferred_element_type=jnp.float32) + b1[...]
    h = _leaky(g1[...] * h * bnc + e1[...])
    h = jnp.dot(h, w2[...], preferred_element_type=jnp.float32) + b2[...]
    h = _leaky(g2[...] * h * bnc + e2[...])
    h = jnp.dot(h, w3[...], preferred_element_type=jnp.float32) + b3[...]
    h = _leaky(g3[...] * h * bnc + e3[...])
    out[...] = jnp.dot(h, cw[...], preferred_element_type=jnp.float32) + cb[...]


def _head(r1, r2, r3, sc, params):
    m = params['mlp']
    lp = params['llm']
    cp = params['clf']
    args = [r1, r2, r3, sc,
            lp['W'], lp['b'].reshape(1, -1),
            m['W1'], m['b1'].reshape(1, -1), m['g1'].reshape(1, -1), m['be1'].reshape(1, -1),
            m['W2'], m['b2'].reshape(1, -1), m['g2'].reshape(1, -1), m['be2'].reshape(1, -1),
            m['W3'], m['b3'].reshape(1, -1), m['g3'].reshape(1, -1), m['be3'].reshape(1, -1),
            cp['W'], cp['b'].reshape(1, -1)]
    return pl.pallas_call(
        _head_body,
        out_shape=jax.ShapeDtypeStruct((1, 2), jnp.float32),
    )(*args)


def _sc_seg_rows(y_flat, src, dst_flat, desc_per_core, nout):
    zsrc = jnp.zeros((64, 128), jnp.float32)
    mesh = plsc.VectorSubcoreMesh(core_axis_name="c", subcore_axis_name="s")

    @functools.partial(
        pl.kernel, mesh=mesh,
        out_type=jax.ShapeDtypeStruct((nout * NPAD, 128), jnp.float32),
        scratch_types=[
            pltpu.VMEM((KB,), jnp.int32),
            pltpu.VMEM((KB,), jnp.int32),
            pltpu.VMEM((KB, 128), jnp.float32),
            pltpu.VMEM((64, 128), jnp.float32),
            pltpu.VMEM_SHARED((NPAD, 128), jnp.float32),
            pltpu.SemaphoreType.DMA,
        ])
    def k(y_ref, src_ref, dst_ref, zsrc_ref, out_ref, idxv, dstv, rows, zbuf,
          acc, sem):
        sub = lax.axis_index("s")
        core = lax.axis_index("c")
        pltpu.sync_copy(zsrc_ref, zbuf)
        for ci, dl in enumerate(desc_per_core):
            @pl.when(core == ci)
            def _(dl=dl):
                for (yoff, doff, elo, en, ooff) in dl:
                    for z in range(RPT // 64):
                        pltpu.sync_copy(zbuf, acc.at[pl.ds(sub * RPT + z * 64, 64)])
                    plsc.subcore_barrier()
                    per_tile = en // NSUB
                    tile_lo = elo + sub * per_tile

                    def blk(j, carry, yoff=yoff, doff=doff, tile_lo=tile_lo):
                        base = tile_lo + j * KB
                        pltpu.sync_copy(src_ref.at[pl.ds(base, KB)], idxv)
                        pltpu.sync_copy(dst_ref.at[pl.ds(doff + base, KB)], dstv)
                        if yoff:
                            for t in range(KB // 16):
                                idxv[pl.ds(16 * t, 16)] = (
                                    idxv[pl.ds(16 * t, 16)] + yoff)
                        pltpu.async_copy(y_ref.at[idxv], rows, sem).wait()
                        pltpu.sync_copy(rows, acc.at[dstv], add=True)
                        return carry

                    lax.fori_loop(0, per_tile // KB, blk, 0)
                    plsc.subcore_barrier()
                    pltpu.sync_copy(acc.at[pl.ds(sub * RPT, RPT)],
                                    out_ref.at[pl.ds(ooff + sub * RPT, RPT)])
                    plsc.subcore_barrier()

    return k(y_flat, src, dst_flat, zsrc)


def _chunkT(x, nc):
    return x.reshape(NPAD, nc, 128).transpose(1, 0, 2).reshape(nc * NPAD, 128)


def _unchunkT(o, nc):
    return o.reshape(nc, NPAD, 128).transpose(1, 0, 2).reshape(NPAD, nc * 128)


def _seg_chunks(y, srcp, dstp, nc):
    desc = [[(c * NPAD, 0, 0, EP, c * NPAD) for c in range(ci, nc, 2)]
            for ci in range(2)]
    out = _sc_seg_rows(_chunkT(y, nc) if nc > 1 else y, srcp, dstp, desc, nc)
    return _unchunkT(out, nc) if nc > 1 else out


def _seg_scalar(vals, dstp):
    return jnp.zeros((NPAD,), jnp.float32).at[dstp].add(vals)


def _degree(dstp):
    return _seg_scalar(jnp.ones((EP,), jnp.float32), dstp) + 1.0


def _pool(h, srcp, dstp, alive, pp, k):
    sr = _linear([(h, jnp.concatenate([pp['Wrel'], pp['Wroot']], axis=1))],
                 None, False)
    aggs = _seg_scalar(sr[srcp, 0], dstp)
    score = aggs + pp['brel'][0] + sr[:, 1]
    smask = jnp.where(alive > 0, score, -jnp.inf)
    _, perm = lax.top_k(smask, k)
    sel = jnp.zeros((NPAD,), jnp.float32).at[perm].set(1.0)
    gfac = jnp.tanh(score) * sel
    return h * gfac[:, None], sel


def _attn(h, alive, gw, gb):
    gate = _linear([(h, gw)], gb, False)[:, 0]
    g = jnp.where(alive > 0, gate, -jnp.inf)
    e = jnp.exp(g - jnp.max(g)) * alive
    a = e / jnp.sum(e)
    return _linear([(a.reshape(1, NPAD), h)], None, False)


def _pad_edges(src, dst):
    pad = EP - E0
    srcp = jnp.concatenate([src, jnp.zeros((pad,), jnp.int32)])
    dstb = jnp.concatenate([dst, jnp.full((pad,), TRASH, jnp.int32)])
    return srcp, dstb


def _gcn_branch(x, src0, dst0, p):
    srcp, dstb = _pad_edges(src0, dst0)
    alive = (jnp.arange(NPAD) < N0).astype(jnp.float32)
    dstp = dstb
    dis = lax.rsqrt(_degree(dstp))
    y = x * dis[:, None]
    out = _sc_seg_rows(y, srcp, dstp,
                       [[(0, 0, 0, EP // 2, 0)],
                        [(0, 0, EP // 2, EP // 2, NPAD)]], 2)
    raw = out[:NPAD] + out[NPAD:]
    z = raw * dis[:, None] + x * (dis * dis)[:, None]
    h = _linear([(z, p['W1'])], p['b1'], True)
    h, alive = _pool(h, srcp, dstp, alive, p['pool1'], 8000)
    ok = alive[srcp] * alive[dstb]
    dstp = jnp.where(ok > 0, dstb, TRASH)
    dis = lax.rsqrt(_degree(dstp))
    xw = _linear([(h, p['W2'])], None, False)
    raw = _seg_chunks(xw * dis[:, None], srcp, dstp, 4)
    h = _leaky(raw * dis[:, None] + xw * (dis * dis)[:, None] + p['b2'][None, :])
    h, alive = _pool(h, srcp, dstp, alive, p['pool2'], 6400)
    ok = alive[srcp] * alive[dstb]
    dstp = jnp.where(ok > 0, dstb, TRASH)
    dis = lax.rsqrt(_degree(dstp))
    xw = _linear([(h, p['W3'])], None, False)
    raw = _seg_chunks(xw * dis[:, None], srcp, dstp, 2)
    h = _leaky(raw * dis[:, None] + xw * (dis * dis)[:, None] + p['b3'][None, :])
    return _attn(h, alive, p['gate_W'], p['gate_b'])


def _rgcn_seg(y, srcp, rel_flat, nc_per_rel):
    nc = 2 * nc_per_rel
    desc = [[(j * NPAD, (j // nc_per_rel) * EP, 0, EP, j * NPAD)
             for j in range(ci, nc, 2)] for ci in range(2)]
    out = _sc_seg_rows(y, srcp, rel_flat, desc, nc)
    return _unchunkT(out, nc)


def _pdg_branch(x, src0, dst0, et0, p):
    srcp, dstb = _pad_edges(src0, dst0)
    et = jnp.concatenate([et0, jnp.zeros((EP - E0,), jnp.int32)])
    alive = (jnp.arange(NPAD) < N0).astype(jnp.float32)
    dstp = dstb
    rel = [jnp.where(et == r, dstb, TRASH) for r in range(2)]
    relf = jnp.concatenate(rel)
    cnt = [jnp.maximum(_seg_scalar(jnp.ones((EP,), jnp.float32), rel[r]), 1.0)
           for r in range(2)]
    out = _sc_seg_rows(x, srcp, relf,
                       [[(0, 0, 0, EP, 0)], [(0, EP, 0, EP, NPAD)]], 2)
    parts = [(x, p['Wroot1'])]
    for r in range(2):
        parts.append((out[r * NPAD:(r + 1) * NPAD] / cnt[r][:, None],
                      p['Wr1'][r]))
    h = _linear(parts, p['b1'], True)
    h, alive = _pool(h, srcp, dstp, alive, p['pool1'], 8000)
    for li, (Wr, Wroot, b, dout, kpool) in enumerate((
            (p['Wr2'], p['Wroot2'], p['b2'], 512, 6400),
            (p['Wr3'], p['Wroot3'], p['b3'], 256, None))):
        ok = alive[srcp] * alive[dstb]
        dstp = jnp.where(ok > 0, dstb, TRASH)
        rel = [jnp.where((ok > 0) & (et == r), dstb, TRASH) for r in range(2)]
        relf = jnp.concatenate(rel)
        cnt = [jnp.maximum(_seg_scalar(jnp.ones((EP,), jnp.float32), rel[r]),
                           1.0) for r in range(2)]
        xw = _linear([(h, jnp.concatenate([Wr[0], Wr[1]], axis=1))], None,
                     False)
        nc_per_rel = dout // 128
        raw = _rgcn_seg(_chunkT(xw, 2 * nc_per_rel), srcp, relf, nc_per_rel)
        root = _linear([(h, Wroot)], b, False)
        h = _leaky(root + raw[:, :dout] / cnt[0][:, None]
                   + raw[:, dout:] / cnt[1][:, None])
        if kpool is not None:
            h, alive = _pool(h, srcp, dstp, alive, p['pool2'], kpool)
    return _attn(h, alive, p['gate_W'], p['gate_b'])


def _pad_rows(x):
    return jnp.pad(x, ((0, NPAD - N0), (0, 0)))


def kernel(ast_x, ast_edge_index, cfg_x, cfg_edge_index, pdg_x,
           pdg_edge_index, pdg_edge_type, source_code, params):
    r1 = _gcn_branch(_pad_rows(ast_x), ast_edge_index[0], ast_edge_index[1],
                     params['ast'])
    r2 = _gcn_branch(_pad_rows(cfg_x), cfg_edge_index[0], cfg_edge_index[1],
                     params['cfg'])
    et = (pdg_edge_type != jnp.min(pdg_edge_type)).astype(jnp.int32)
    r3 = _pdg_branch(_pad_rows(pdg_x), pdg_edge_index[0], pdg_edge_index[1],
                     et, params['pdg'])
    return _head(r1, r2, r3, source_code, params)


# speedup vs baseline: 6.3804x; 6.3804x over previous
import functools
import math

import jax
import jax.numpy as jnp
from jax import lax
from jax.experimental import pallas as pl
from jax.experimental.pallas import tpu as pltpu
from jax.experimental.pallas import tpu_sc as plsc

N0 = 10000
E0 = 160000
NPAD = 10240
TRASH = N0
EP = 163840
BM = 256
KB = 128
CW = 128
NSUB = 16
RPT = NPAD // NSUB
EPT = EP // 32

_MESH = dict(core_axis_name="c", subcore_axis_name="s")


def _leaky(x):
    return jnp.where(x >= 0, x, 0.01 * x)


def _linear_body(*refs, nparts, act):
    out_ref = refs[-1]
    bias_ref = refs[-2]
    acc = jnp.zeros(out_ref.shape, jnp.float32)
    for i in range(nparts):
        a = refs[2 * i][...]
        w = refs[2 * i + 1][...]
        acc += jnp.dot(a, w, preferred_element_type=jnp.float32)
    acc = acc + bias_ref[...]
    if act:
        acc = _leaky(acc)
    out_ref[...] = acc


def _linear(parts, bias, act):
    rows = parts[0][0].shape[0]
    dout = parts[0][1].shape[1]
    bm = BM if rows % BM == 0 else rows
    grid = (rows // bm,)
    in_specs = []
    args = []
    for a, w in parts:
        di = a.shape[1]
        in_specs.append(pl.BlockSpec((bm, di), lambda i: (i, 0)))
        in_specs.append(pl.BlockSpec((di, dout), lambda i: (0, 0)))
        args += [a, w]
    b2 = jnp.zeros((1, dout), jnp.float32) if bias is None else bias.reshape(1, dout)
    in_specs.append(pl.BlockSpec((1, dout), lambda i: (0, 0)))
    args.append(b2)
    return pl.pallas_call(
        functools.partial(_linear_body, nparts=len(parts), act=act),
        grid=grid,
        in_specs=in_specs,
        out_specs=pl.BlockSpec((bm, dout), lambda i: (i, 0)),
        out_shape=jax.ShapeDtypeStruct((rows, dout), jnp.float32),
    )(*args)


def _head_body(r1, r2, r3, sc, lw, lb, w1, b1, g1, e1, w2, b2, g2, e2,
               w3, b3, g3, e3, cw, cb, out):
    bnc = 1.0 / math.sqrt(1.0 + 1e-5)
    llm = jnp.dot(sc[...], lw[...], preferred_element_type=jnp.float32) + lb[...]
    h = jnp.concatenate([r1[...], r2[...], r3[...], llm], axis=1)
    h = jnp.dot(h, w1[...], preferred_element_type=jnp.float32) + b1[...]
    h = _leaky(g1[...] * h * bnc + e1[...])
    h = jnp.dot(h, w2[...], preferred_element_type=jnp.float32) + b2[...]
    h = _leaky(g2[...] * h * bnc + e2[...])
    h = jnp.dot(h, w3[...], preferred_element_type=jnp.float32) + b3[...]
    h = _leaky(g3[...] * h * bnc + e3[...])
    out[...] = jnp.dot(h, cw[...], preferred_element_type=jnp.float32) + cb[...]


def _head(r1, r2, r3, sc, params):
    m = params['mlp']
    lp = params['llm']
    cp = params['clf']
    args = [r1, r2, r3, sc,
            lp['W'], lp['b'].reshape(1, -1),
            m['W1'], m['b1'].reshape(1, -1), m['g1'].reshape(1, -1), m['be1'].reshape(1, -1),
            m['W2'], m['b2'].reshape(1, -1), m['g2'].reshape(1, -1), m['be2'].reshape(1, -1),
            m['W3'], m['b3'].reshape(1, -1), m['g3'].reshape(1, -1), m['be3'].reshape(1, -1),
            cp['W'], cp['b'].reshape(1, -1)]
    return pl.pallas_call(
        _head_body,
        out_shape=jax.ShapeDtypeStruct((1, 2), jnp.float32),
    )(*args)


def _sc_seg_rows(y_flat, src_flat, dst2d, nout, ndesc, en, fns, add_yoff):
    zsrc = jnp.zeros((16, CW), jnp.float32)
    mesh = plsc.VectorSubcoreMesh(**_MESH)
    nblk = en // NSUB // KB
    pet = nblk * KB
    BB = 40 if nblk % 40 == 0 else nblk
    BE = BB * KB

    @functools.partial(
        pl.kernel, mesh=mesh,
        out_type=jax.ShapeDtypeStruct((nout * NPAD, CW), jnp.float32),
        compiler_params=pltpu.CompilerParams(needs_layout_passes=False),
        scratch_types=[
            pltpu.VMEM((BE,), jnp.int32),
            pltpu.VMEM((BB, KB), jnp.int32),
            pltpu.VMEM((KB, CW), jnp.float32),
            pltpu.VMEM((KB, CW), jnp.float32),
            pltpu.VMEM((16, CW), jnp.float32),
            pltpu.VMEM_SHARED((NPAD, CW), jnp.float32),
            pltpu.SemaphoreType.DMA,
            pltpu.SemaphoreType.DMA,
        ])
    def k(y_ref, src_ref, dst_ref, zsrc_ref, out_ref, srcv, dstb, rows0,
          rows1, zbuf, acc, gA, gB):
        sub = lax.axis_index("s")
        core = lax.axis_index("c")
        pltpu.sync_copy(zsrc_ref, zbuf)
        for i in range(ndesc // 2):
            d = 2 * i + core
            yoff, drow, elo = fns(d)
            ooff = d * NPAD
            for z in range(RPT // 16):
                pltpu.sync_copy(zbuf, acc.at[pl.ds(sub * RPT + z * 16, 16)])
            plsc.subcore_barrier()
            for bb in range(nblk // BB):
                pltpu.sync_copy(
                    src_ref.at[pl.ds(elo + sub * pet + bb * BE, BE)], srcv)
                pltpu.sync_copy(
                    dst_ref.at[pl.ds(drow + sub * nblk + bb * BB, BB)], dstb)
                if add_yoff:
                    def adj(j, c2):
                        srcv[pl.ds(j * 16, 16)] = (
                            srcv[pl.ds(j * 16, 16)] + yoff)
                        return c2
                    lax.fori_loop(0, BE // 16, adj, 0)
                pltpu.async_copy(y_ref.at[srcv.at[pl.ds(0, KB)]], rows0, gA)

                def pair(m, c2):
                    pltpu.make_async_copy(y_ref.at[pl.ds(0, KB)], rows0,
                                          gA).wait()
                    pltpu.async_copy(
                        y_ref.at[srcv.at[pl.ds((2 * m + 1) * KB, KB)]],
                        rows1, gB)
                    pltpu.sync_copy(rows0, acc.at[dstb.at[2 * m]], add=True)
                    pltpu.make_async_copy(y_ref.at[pl.ds(0, KB)], rows1,
                                          gB).wait()
                    nxt = jnp.minimum(2 * m + 2, BB - 1) * KB
                    pltpu.async_copy(y_ref.at[srcv.at[pl.ds(nxt, KB)]],
                                     rows0, gA)
                    pltpu.sync_copy(rows1, acc.at[dstb.at[2 * m + 1]],
                                    add=True)
                    return c2

                lax.fori_loop(0, BB // 2, pair, 0)
                pltpu.make_async_copy(y_ref.at[pl.ds(0, KB)], rows0,
                                      gA).wait()
            plsc.subcore_barrier()
            pltpu.sync_copy(acc.at[pl.ds(sub * RPT, RPT)],
                            out_ref.at[pl.ds(ooff + sub * RPT, RPT)])
            plsc.subcore_barrier()

    return k(y_flat, src_flat, dst2d, zsrc)


def _chunkT(x):
    nc = x.shape[1] // CW
    return x.reshape(NPAD, nc, CW).transpose(1, 0, 2).reshape(nc * NPAD, CW)


def _unchunkT(o, nc):
    return o.reshape(nc, NPAD, CW).transpose(1, 0, 2).reshape(NPAD, nc * CW)


ERB = EP // KB


def _fill16(ref, n, val):
    for i in range(n // 16):
        ref[pl.ds(i * 16, 16)] = jnp.full((16,), val, jnp.float32)


def _prep_call(src, dst, sel, et=None):
    nrel = 0 if et is None else 2
    nd = 1 + nrel
    ncnt = max(nrel, 1)
    mesh = plsc.VectorSubcoreMesh(**_MESH)
    scratch = [
        pltpu.VMEM((NPAD,), jnp.float32),
        pltpu.VMEM((EPT,), jnp.int32),
        pltpu.VMEM((EPT,), jnp.int32),
        pltpu.VMEM((128,), jnp.int32),
        pltpu.VMEM((128,), jnp.float32),
        pltpu.VMEM((RPT,), jnp.float32),
    ] + [pltpu.VMEM((EPT,), jnp.int32) for _ in range(nd)] \
      + ([pltpu.VMEM((EPT,), jnp.int32)] if et is not None else []) \
      + [pltpu.VMEM_SHARED((NPAD,), jnp.float32) for _ in range(ncnt)]

    @functools.partial(
        pl.kernel, mesh=mesh,
        out_type=(jax.ShapeDtypeStruct((nd * EP,), jnp.int32),
                  jax.ShapeDtypeStruct((ncnt * 2 * NPAD,), jnp.float32)),
        compiler_params=pltpu.CompilerParams(needs_layout_passes=False),
        scratch_types=scratch)
    def k(src_ref, dst_ref, sel_ref, *rest):
        if et is None:
            (dstp_out, cnt_out, selv, srcv, dstv, didx, ones, zv,
             dp0) = rest[:9]
            accs = rest[9:]
            dps = [dp0]
            etv = None
        else:
            (et_ref, dstp_out, cnt_out, selv, srcv, dstv, didx, ones, zv,
             dp0, dp1, dp2, etv) = rest[:13]
            accs = rest[13:]
            dps = [dp0, dp1, dp2]
        sub = lax.axis_index("s")
        core = lax.axis_index("c")
        wid = core * NSUB + sub
        base = wid * EPT
        _fill16(ones, 128, 1.0)
        _fill16(zv, RPT, 0.0)
        pltpu.sync_copy(sel_ref, selv)
        pltpu.sync_copy(src_ref.at[pl.ds(base, EPT)], srcv)
        pltpu.sync_copy(dst_ref.at[pl.ds(base, EPT)], dstv)
        if et is not None:
            pltpu.sync_copy(et_ref.at[pl.ds(base, EPT)], etv)
        for a in accs:
            pltpu.sync_copy(zv, a.at[pl.ds(sub * RPT, RPT)])
        plsc.subcore_barrier()

        def it(i, c):
            s16 = srcv[pl.ds(i * 16, 16)]
            d16 = dstv[pl.ds(i * 16, 16)]
            ok = (plsc.load_gather(selv, [s16])
                  * plsc.load_gather(selv, [d16])) > 0.0
            t16 = jnp.full((16,), TRASH, jnp.int32)
            dps[0][pl.ds(i * 16, 16)] = jnp.where(ok, d16, t16)
            if et is not None:
                e16 = etv[pl.ds(i * 16, 16)]
                for r in range(2):
                    dps[1 + r][pl.ds(i * 16, 16)] = jnp.where(
                        ok & (e16 == r), d16, t16)
            return c

        lax.fori_loop(0, EPT // 16, it, 0)
        cnt_src = dps[1:] if et is not None else dps[:1]
        for r, dp in enumerate(cnt_src):
            for j in range(EPT // 128):
                for t in range(8):
                    didx[pl.ds(t * 16, 16)] = dp[pl.ds(j * 128 + t * 16, 16)]
                pltpu.sync_copy(ones, accs[r].at[didx], add=True)
        for d, dp in enumerate(dps):
            pltpu.sync_copy(dp, dstp_out.at[pl.ds(d * EP + base, EPT)])
        plsc.subcore_barrier()
        for r in range(ncnt):
            pltpu.sync_copy(
                accs[r].at[pl.ds(sub * RPT, RPT)],
                cnt_out.at[pl.ds((r * 2 + core) * NPAD + sub * RPT, RPT)])

    args = (src, dst, sel) if et is None else (src, dst, sel, et)
    dstp_all, cnt = k(*args)
    cnts = cnt.reshape(ncnt, 2, NPAD).sum(axis=1)
    return dstp_all, cnts


def _pool_agg(src, dstp, srel):
    mesh = plsc.VectorSubcoreMesh(**_MESH)

    @functools.partial(
        pl.kernel, mesh=mesh,
        out_type=jax.ShapeDtypeStruct((2 * NPAD,), jnp.float32),
        compiler_params=pltpu.CompilerParams(needs_layout_passes=False),
        scratch_types=[
            pltpu.VMEM((NPAD,), jnp.float32),
            pltpu.VMEM((EPT,), jnp.int32),
            pltpu.VMEM((EPT,), jnp.int32),
            pltpu.VMEM((EPT,), jnp.float32),
            pltpu.VMEM((128,), jnp.int32),
            pltpu.VMEM((128,), jnp.float32),
            pltpu.VMEM((RPT,), jnp.float32),
            pltpu.VMEM_SHARED((NPAD,), jnp.float32),
        ])
    def k(src_ref, dst_ref, tab_ref, out_ref, tabv, srcv, dstv, vals, didx,
          v128, zv, acc):
        sub = lax.axis_index("s")
        core = lax.axis_index("c")
        base = (core * NSUB + sub) * EPT
        _fill16(zv, RPT, 0.0)
        pltpu.sync_copy(tab_ref, tabv)
        pltpu.sync_copy(src_ref.at[pl.ds(base, EPT)], srcv)
        pltpu.sync_copy(dst_ref.at[pl.ds(base, EPT)], dstv)
        pltpu.sync_copy(zv, acc.at[pl.ds(sub * RPT, RPT)])
        plsc.subcore_barrier()

        def it(i, c):
            s16 = srcv[pl.ds(i * 16, 16)]
            vals[pl.ds(i * 16, 16)] = plsc.load_gather(tabv, [s16])
            return c

        lax.fori_loop(0, EPT // 16, it, 0)
        for j in range(EPT // 128):
            for t in range(8):
                didx[pl.ds(t * 16, 16)] = dstv[pl.ds(j * 128 + t * 16, 16)]
                v128[pl.ds(t * 16, 16)] = vals[pl.ds(j * 128 + t * 16, 16)]
            pltpu.sync_copy(v128, acc.at[didx], add=True)
        plsc.subcore_barrier()
        pltpu.sync_copy(acc.at[pl.ds(sub * RPT, RPT)],
                        out_ref.at[pl.ds(core * NPAD + sub * RPT, RPT)])

    out = k(src, dstp, srel)
    return out[:NPAD] + out[NPAD:]


def _pool(h, srcp, dstp, alive, pp, kk):
    sr = _linear([(h, jnp.concatenate([pp['Wrel'], pp['Wroot']], axis=1))],
                 None, False)
    aggs = _pool_agg(srcp, dstp, sr[:, 0])
    score = aggs + pp['brel'][0] + sr[:, 1]
    smask = jnp.where(alive > 0, score, -jnp.inf)
    _, perm = lax.top_k(smask, kk)
    sel = jnp.zeros((NPAD,), jnp.float32).at[perm].set(1.0)
    gfac = jnp.tanh(score) * sel
    return h * gfac[:, None], sel


def _attn(h, alive, gw, gb):
    gate = _linear([(h, gw)], gb, False)[:, 0]
    g = jnp.where(alive > 0, gate, -jnp.inf)
    e = jnp.exp(g - jnp.max(g)) * alive
    a = e / jnp.sum(e)
    return _linear([(a.reshape(1, NPAD), h)], None, False)


def _pad_edges(src, dst):
    pad = EP - E0
    srcp = jnp.concatenate([src, jnp.zeros((pad,), jnp.int32)])
    dstb = jnp.concatenate([dst, jnp.full((pad,), TRASH, jnp.int32)])
    return srcp, dstb


def _gcn_branch(x, src0, dst0, p):
    srcp, dstb = _pad_edges(src0, dst0)
    alive = (jnp.arange(NPAD) < N0).astype(jnp.float32)
    dstp, deg = _prep_call(srcp, dstb, alive)
    dis = lax.rsqrt(deg[0] + 1.0)
    y = x * dis[:, None]
    out = _sc_seg_rows(y, srcp, dstp.reshape(-1, KB), 2, 2, EP // 2,
                       lambda d: (0, d * (ERB // 2), d * (EP // 2)), False)
    raw = out[:NPAD] + out[NPAD:]
    z = raw * dis[:, None] + x * (dis * dis)[:, None]
    h = _linear([(z, p['W1'])], p['b1'], True)
    h, alive = _pool(h, srcp, dstp, alive, p['pool1'], 8000)
    for (W, b, pp, kk) in ((p['W2'], p['b2'], p['pool2'], 6400),
                           (p['W3'], p['b3'], None, None)):
        nc = W.shape[1] // CW
        dstp, deg = _prep_call(srcp, dstb, alive)
        dis = lax.rsqrt(deg[0] + 1.0)
        xw = _linear([(h, W)], None, False)
        y = _chunkT(xw * dis[:, None])
        out = _sc_seg_rows(y, srcp, dstp.reshape(-1, KB), nc, nc, EP,
                           lambda d: (d * NPAD, 0, 0), True)
        raw = _unchunkT(out, nc)
        h = _leaky(raw * dis[:, None] + xw * (dis * dis)[:, None] + b[None, :])
        if pp is not None:
            h, alive = _pool(h, srcp, dstp, alive, pp, kk)
    return _attn(h, alive, p['gate_W'], p['gate_b'])


def _pdg_branch(x, src0, dst0, et0, p):
    srcp, dstb = _pad_edges(src0, dst0)
    et = jnp.concatenate([et0, jnp.zeros((EP - E0,), jnp.int32)])
    alive = (jnp.arange(NPAD) < N0).astype(jnp.float32)
    dstp_all, cnts = _prep_call(srcp, dstb, alive, et)
    dstp = dstp_all[:EP]
    out = _sc_seg_rows(x, srcp, dstp_all.reshape(-1, KB), 2, 2, EP,
                       lambda d: (0, (1 + d) * ERB, 0), False)
    parts = [(x, p['Wroot1'])]
    for r in range(2):
        parts.append((out[r * NPAD:(r + 1) * NPAD]
                      / jnp.maximum(cnts[r], 1.0)[:, None], p['Wr1'][r]))
    h = _linear(parts, p['b1'], True)
    h, alive = _pool(h, srcp, dstp, alive, p['pool1'], 8000)
    for (Wr, Wroot, b, dout, kk) in ((p['Wr2'], p['Wroot2'], p['b2'], 512,
                                      6400),
                                     (p['Wr3'], p['Wroot3'], p['b3'], 256,
                                      None)):
        dstp_all, cnts = _prep_call(srcp, dstb, alive, et)
        dstp = dstp_all[:EP]
        xw = _linear([(h, jnp.concatenate([Wr[0], Wr[1]], axis=1))], None,
                     False)
        ncr = dout // CW
        nc = 2 * ncr
        y = _chunkT(xw)
        out = _sc_seg_rows(y, srcp, dstp_all.reshape(-1, KB), nc, nc, EP,
                           lambda d: (d * NPAD, (1 + d // ncr) * ERB, 0),
                           True)
        raw = _unchunkT(out, nc)
        root = _linear([(h, Wroot)], b, False)
        h = _leaky(root + raw[:, :dout] / jnp.maximum(cnts[0], 1.0)[:, None]
                   + raw[:, dout:] / jnp.maximum(cnts[1], 1.0)[:, None])
        if kk is not None:
            h, alive = _pool(h, srcp, dstp, alive, p['pool2'], kk)
    return _attn(h, alive, p['gate_W'], p['gate_b'])


def _pad_rows(x):
    return jnp.pad(x, ((0, NPAD - N0), (0, 0)))


def kernel(ast_x, ast_edge_index, cfg_x, cfg_edge_index, pdg_x,
           pdg_edge_index, pdg_edge_type, source_code, params):
    r1 = _gcn_branch(_pad_rows(ast_x), ast_edge_index[0], ast_edge_index[1],
                     params['ast'])
    r2 = _gcn_branch(_pad_rows(cfg_x), cfg_edge_index[0], cfg_edge_index[1],
                     params['cfg'])
    et = (pdg_edge_type != jnp.min(pdg_edge_type)).astype(jnp.int32)
    r3 = _pdg_branch(_pad_rows(pdg_x), pdg_edge_index[0], pdg_edge_index[1],
                     et, params['pdg'])
    return _head(r1, r2, r3, source_code, params)
